# Initial kernel scaffold; baseline (speedup 1.0000x reference)
import jax
import jax.numpy as jnp
from jax.experimental import pallas as pl


def kernel(x, edge_index, edge_attr, batch, params):
    raise NotImplementedError("write your pallas kernel here")


# speedup vs baseline: 1.6954x; 1.6954x over previous
import functools

import jax
import jax.numpy as jnp
from jax import lax
from jax.experimental import pallas as pl
from jax.experimental.pallas import tpu as pltpu
from jax.experimental.pallas import tpu_sc as plsc

_NC = 2
_NS = 16


@functools.lru_cache(maxsize=None)
def _build_sc_agg(N, E, D, K):
    NW = _NC * _NS
    EPT = E // NW
    NCH = EPT // K
    NPT = (N // (8 * _NS)) * 8
    REM = N - NPT * _NS
    assert EPT * NW == E and NCH * K == EPT and REM % 8 == 0

    mesh = plsc.VectorSubcoreMesh(core_axis_name="c", subcore_axis_name="s")

    @functools.partial(
        pl.kernel,
        mesh=mesh,
        out_type=jax.ShapeDtypeStruct((_NC * N, D), jnp.float32),
        scratch_types=[
            pltpu.VMEM((K,), jnp.int32),
            pltpu.VMEM((K,), jnp.int32),
            pltpu.VMEM((K, D), jnp.float32),
            pltpu.VMEM((K, D), jnp.float32),
            pltpu.VMEM_SHARED((N, D), jnp.float32),
            pltpu.SemaphoreType.DMA,
        ],
    )
    def sc_agg(h_hbm, e_hbm, src_hbm, dst_hbm, zero_hbm, out_hbm,
               idx_s, idx_d, hbuf, ebuf, agg_sh, sem):
        c = lax.axis_index("c")
        s = lax.axis_index("s")
        pltpu.sync_copy(zero_hbm.at[pl.ds(s * NPT, NPT)],
                        agg_sh.at[pl.ds(s * NPT, NPT)])
        if REM:
            @pl.when(s == _NS - 1)
            def _():
                pltpu.sync_copy(zero_hbm.at[pl.ds(_NS * NPT, REM)],
                                agg_sh.at[pl.ds(_NS * NPT, REM)])
        plsc.subcore_barrier()

        base = c * (E // _NC) + s * EPT

        def chunk(t, carry):
            off = base + t * K
            pltpu.sync_copy(src_hbm.at[pl.ds(off, K)], idx_s)
            pltpu.sync_copy(dst_hbm.at[pl.ds(off, K)], idx_d)
            pltpu.sync_copy(e_hbm.at[pl.ds(off, K)], ebuf)
            pltpu.async_copy(h_hbm.at[idx_s], hbuf, sem).wait()

            def row(j, cc):
                for v in range(D // 16):
                    sl = pl.ds(v * 16, 16)
                    hbuf[j, sl] = jnp.maximum(hbuf[j, sl] + ebuf[j, sl], 0.0)
                return cc

            lax.fori_loop(0, K, row, 0, unroll=2)
            pltpu.sync_copy(hbuf, agg_sh.at[idx_d], add=True)
            return carry

        lax.fori_loop(0, NCH, chunk, 0)
        plsc.subcore_barrier()
        pltpu.sync_copy(agg_sh.at[pl.ds(s * NPT, NPT)],
                        out_hbm.at[pl.ds(c * N + s * NPT, NPT)])
        if REM:
            @pl.when(s == _NS - 1)
            def _():
                pltpu.sync_copy(agg_sh.at[pl.ds(_NS * NPT, REM)],
                                out_hbm.at[pl.ds(c * N + _NS * NPT, REM)])

    return sc_agg


@functools.lru_cache(maxsize=None)
def _build_edge_embed(E, DE, D, EB):
    grid = (E // EB,)

    def body(ea_ref, w_ref, b_ref, o0, o1, o2, o3):
        ea = ea_ref[...]
        outs = (o0, o1, o2, o3)
        for i in range(4):
            outs[i][...] = (
                jnp.dot(ea, w_ref[i], preferred_element_type=jnp.float32)
                + b_ref[i, :][None, :]
            )

    return pl.pallas_call(
        body,
        grid=grid,
        in_specs=[
            pl.BlockSpec((EB, DE), lambda i: (i, 0)),
            pl.BlockSpec((4, DE, D), lambda i: (0, 0, 0)),
            pl.BlockSpec((4, D), lambda i: (0, 0)),
        ],
        out_specs=[pl.BlockSpec((EB, D), lambda i: (i, 0))] * 4,
        out_shape=[jax.ShapeDtypeStruct((E, D), jnp.float32)] * 4,
    )


@functools.lru_cache(maxsize=None)
def _build_dense(N, D):
    def body(h_ref, ap_ref, w1_ref, b1_ref, w2_ref, b2_ref, g_ref, bt_ref,
             out_ref):
        ap = ap_ref[...]
        z = h_ref[...] + ap[:N] + ap[N:]
        z = jnp.dot(z, w1_ref[...], preferred_element_type=jnp.float32)
        z = jnp.maximum(z + b1_ref[...], 0.0)
        z = jnp.dot(z, w2_ref[...], preferred_element_type=jnp.float32)
        z = z + b2_ref[...]
        mu = jnp.mean(z, axis=0, keepdims=True)
        var = jnp.mean(z * z, axis=0, keepdims=True) - mu * mu
        zn = (z - mu) * lax.rsqrt(var + 1e-5) * g_ref[...] + bt_ref[...]
        out_ref[...] = jnp.maximum(zn, 0.0)

    return pl.pallas_call(
        body, out_shape=jax.ShapeDtypeStruct((N, D), jnp.float32))


@functools.lru_cache(maxsize=None)
def _build_pool(N, D, G):
    def body(h_ref, b_ref, w1_ref, b1_ref, w2_ref, b2_ref, w3_ref, b3_ref,
             out_ref):
        gids = lax.broadcasted_iota(jnp.int32, (G, N), 0)
        onehot = (gids == b_ref[...]).astype(jnp.float32)
        sums = jnp.dot(onehot, h_ref[...], preferred_element_type=jnp.float32)
        cnt = jnp.sum(onehot, axis=1, keepdims=True)
        gm = sums / jnp.maximum(cnt, 1.0)
        z = jnp.dot(gm, w1_ref[...], preferred_element_type=jnp.float32)
        z = z + b1_ref[...]
        z = jnp.where(z > 0, z, 0.2 * z)
        z = jnp.dot(z, w2_ref[...], preferred_element_type=jnp.float32)
        z = z + b2_ref[...]
        z = jnp.where(z > 0, z, 0.2 * z)
        z = jnp.dot(z, w3_ref[...], preferred_element_type=jnp.float32)
        out_ref[...] = z + b3_ref[...]

    return pl.pallas_call(
        body, out_shape=jax.ShapeDtypeStruct((G, 1), jnp.float32))


def kernel(x, edge_index, edge_attr, batch, params):
    N, D = x.shape
    E, DE = edge_attr.shape
    G = 64

    p = params
    src = edge_index[0]
    dst = edge_index[1]
    zeros = jnp.zeros((N, D), jnp.float32)

    wstack = jnp.stack([p["We0"], p["We1"], p["We2"], p["We3"]])
    bstack = jnp.stack([p["be0"], p["be1"], p["be2"], p["be3"]])
    e_all = _build_edge_embed(E, DE, D, 8000)(edge_attr, wstack, bstack)

    sc_agg = _build_sc_agg(N, E, D, 80)
    dense = _build_dense(N, D)

    h = x
    for i in range(4):
        agg_p = sc_agg(h, e_all[i], src, dst, zeros)
        h = dense(
            h, agg_p,
            p[f"W1{i}"], p[f"b1{i}"].reshape(1, D),
            p[f"W2{i}"], p[f"b2{i}"].reshape(1, D),
            p[f"g{i}"].reshape(1, D), p[f"bt{i}"].reshape(1, D),
        )

    score = _build_pool(N, D, G)(
        h, batch.reshape(1, N),
        p["Wm1"], p["bm1"].reshape(1, -1),
        p["Wm2"], p["bm2"].reshape(1, -1),
        p["Wm3"], p["bm3"].reshape(1, -1),
    )
    return score


# speedup vs baseline: 2.7280x; 1.6090x over previous
import functools

import jax
import jax.numpy as jnp
from jax import lax
from jax.experimental import pallas as pl
from jax.experimental.pallas import tpu as pltpu
from jax.experimental.pallas import tpu_sc as plsc

_NC = 2
_NS = 16


@functools.lru_cache(maxsize=None)
def _build_sc_agg(N, E, D, K):
    NW = _NC * _NS
    EPT = E // NW
    NCH = EPT // K
    NPT = (N // (8 * _NS)) * 8
    REM = N - NPT * _NS
    assert EPT * NW == E and NCH * K == EPT and REM % 8 == 0 and K % 8 == 0
    assert NCH % 2 == 1 and NCH >= 5
    E2 = E // _NC
    MAXOFF = E - K

    mesh = plsc.VectorSubcoreMesh(core_axis_name="c", subcore_axis_name="s")

    @functools.partial(
        pl.kernel,
        mesh=mesh,
        out_type=jax.ShapeDtypeStruct((_NC * N, D), jnp.float32),
        scratch_types=[
            pltpu.VMEM((K,), jnp.int32), pltpu.VMEM((K,), jnp.int32),
            pltpu.VMEM((K,), jnp.int32), pltpu.VMEM((K,), jnp.int32),
            pltpu.VMEM((K, D), jnp.float32), pltpu.VMEM((K, D), jnp.float32),
            pltpu.VMEM((K, D), jnp.float32), pltpu.VMEM((K, D), jnp.float32),
            pltpu.VMEM_SHARED((N, D), jnp.float32),
        ] + [pltpu.SemaphoreType.DMA] * 10,
    )
    def sc_agg(h_hbm, e_hbm, src_hbm, dst_hbm, zero_hbm, out_hbm,
               is0, is1, id0, id1, hb0, hb1, eb0, eb1, agg_sh,
               ssi0, ssi1, sdi0, sdi1, se0, se1, sg0, sg1, ss0, ss1):
        idx_s = (is0, is1)
        idx_d = (id0, id1)
        hbuf = (hb0, hb1)
        ebuf = (eb0, eb1)
        s_si = (ssi0, ssi1)
        s_di = (sdi0, sdi1)
        s_e = (se0, se1)
        s_g = (sg0, sg1)
        s_s = (ss0, ss1)
        c = lax.axis_index("c")
        s = lax.axis_index("s")

        pltpu.sync_copy(zero_hbm.at[pl.ds(s * NPT, NPT)],
                        agg_sh.at[pl.ds(s * NPT, NPT)])
        if REM:
            @pl.when(s == _NS - 1)
            def _():
                pltpu.sync_copy(zero_hbm.at[pl.ds(_NS * NPT, REM)],
                                agg_sh.at[pl.ds(_NS * NPT, REM)])

        base = c * E2 + s * EPT

        def off_of(t):
            return jnp.minimum(base + t * K, MAXOFF)

        def issue_idx_s(t, j):
            pltpu.async_copy(src_hbm.at[pl.ds(off_of(t), K)], idx_s[j], s_si[j])

        def issue_idx_d(t, j):
            pltpu.async_copy(dst_hbm.at[pl.ds(off_of(t), K)], idx_d[j], s_di[j])

        def issue_e(t, j):
            pltpu.async_copy(e_hbm.at[pl.ds(off_of(t), K)], ebuf[j], s_e[j])

        def wait_idx_s(j):
            pltpu.make_async_copy(src_hbm.at[pl.ds(0, K)], idx_s[j],
                                  s_si[j]).wait()

        def wait_idx_d(j):
            pltpu.make_async_copy(dst_hbm.at[pl.ds(0, K)], idx_d[j],
                                  s_di[j]).wait()

        def wait_e(j):
            pltpu.make_async_copy(e_hbm.at[pl.ds(0, K)], ebuf[j],
                                  s_e[j]).wait()

        def issue_gather(j):
            pltpu.async_copy(h_hbm.at[idx_s[j]], hbuf[j], s_g[j])

        def wait_gather(j):
            pltpu.make_async_copy(h_hbm.at[idx_s[j]], hbuf[j], s_g[j]).wait()

        def issue_scatter(j):
            pltpu.async_copy(hbuf[j], agg_sh.at[idx_d[j]], s_s[j], add=True)

        def wait_scatter(j):
            pltpu.make_async_copy(hbuf[j], agg_sh.at[idx_d[j]], s_s[j]).wait()

        def compute(j):
            def row(r, cc):
                for v in range(D // 16):
                    sl = pl.ds(v * 16, 16)
                    hbuf[j][r, sl] = jnp.maximum(
                        hbuf[j][r, sl] + ebuf[j][r, sl], 0.0)
                return cc
            lax.fori_loop(0, K, row, 0, unroll=4)

        def step(t, j, first=False):
            if not first:
                wait_scatter(1 - j)
            issue_idx_d(t + 1, 1 - j)
            wait_gather(j)
            wait_e(j)
            wait_idx_s(1 - j)
            issue_gather(1 - j)
            compute(j)
            wait_idx_d(j)
            issue_scatter(j)
            issue_idx_s(t + 2, j)
            issue_e(t + 2, j)

        issue_idx_s(0, 0)
        issue_idx_d(0, 0)
        issue_e(0, 0)
        issue_idx_s(1, 1)
        issue_e(1, 1)
        wait_idx_s(0)
        issue_gather(0)
        plsc.subcore_barrier()

        step(0, 0, first=True)

        def pair(t2, cc):
            t = 1 + 2 * t2
            step(t, 1)
            step(t + 1, 0)
            return cc

        lax.fori_loop(0, (NCH - 3) // 2, pair, 0)
        step(NCH - 2, (NCH - 2) & 1)
        step(NCH - 1, (NCH - 1) & 1)

        j_last = (NCH - 1) & 1
        wait_scatter(j_last)
        wait_gather(1 - j_last)
        wait_idx_d(1 - j_last)
        wait_idx_s(j_last)
        wait_e(j_last)
        wait_e(1 - j_last)
        plsc.subcore_barrier()

        pltpu.sync_copy(agg_sh.at[pl.ds(s * NPT, NPT)],
                        out_hbm.at[pl.ds(c * N + s * NPT, NPT)])
        if REM:
            @pl.when(s == _NS - 1)
            def _():
                pltpu.sync_copy(agg_sh.at[pl.ds(_NS * NPT, REM)],
                                out_hbm.at[pl.ds(c * N + _NS * NPT, REM)])

    return sc_agg


@functools.lru_cache(maxsize=None)
def _build_edge_embed(E, DE, D, EB):
    grid = (E // EB,)

    def body(ea_ref, w_ref, b_ref, o0, o1, o2, o3):
        ea = ea_ref[...]
        outs = (o0, o1, o2, o3)
        for i in range(4):
            outs[i][...] = (
                jnp.dot(ea, w_ref[i], preferred_element_type=jnp.float32)
                + b_ref[i, :][None, :]
            )

    return pl.pallas_call(
        body,
        grid=grid,
        in_specs=[
            pl.BlockSpec((EB, DE), lambda i: (i, 0)),
            pl.BlockSpec((4, DE, D), lambda i: (0, 0, 0)),
            pl.BlockSpec((4, D), lambda i: (0, 0)),
        ],
        out_specs=[pl.BlockSpec((EB, D), lambda i: (i, 0))] * 4,
        out_shape=[jax.ShapeDtypeStruct((E, D), jnp.float32)] * 4,
    )


@functools.lru_cache(maxsize=None)
def _build_dense(N, D):
    def body(h_ref, ap_ref, w1_ref, b1_ref, w2_ref, b2_ref, g_ref, bt_ref,
             out_ref):
        ap = ap_ref[...]
        z = h_ref[...] + ap[:N] + ap[N:]
        z = jnp.dot(z, w1_ref[...], preferred_element_type=jnp.float32)
        z = jnp.maximum(z + b1_ref[...], 0.0)
        z = jnp.dot(z, w2_ref[...], preferred_element_type=jnp.float32)
        z = z + b2_ref[...]
        mu = jnp.mean(z, axis=0, keepdims=True)
        var = jnp.mean(z * z, axis=0, keepdims=True) - mu * mu
        zn = (z - mu) * lax.rsqrt(var + 1e-5) * g_ref[...] + bt_ref[...]
        out_ref[...] = jnp.maximum(zn, 0.0)

    return pl.pallas_call(
        body, out_shape=jax.ShapeDtypeStruct((N, D), jnp.float32))


@functools.lru_cache(maxsize=None)
def _build_pool(N, D, G):
    def body(h_ref, b_ref, w1_ref, b1_ref, w2_ref, b2_ref, w3_ref, b3_ref,
             out_ref):
        gids = lax.broadcasted_iota(jnp.int32, (G, N), 0)
        onehot = (gids == b_ref[...]).astype(jnp.float32)
        sums = jnp.dot(onehot, h_ref[...], preferred_element_type=jnp.float32)
        cnt = jnp.sum(onehot, axis=1, keepdims=True)
        gm = sums / jnp.maximum(cnt, 1.0)
        z = jnp.dot(gm, w1_ref[...], preferred_element_type=jnp.float32)
        z = z + b1_ref[...]
        z = jnp.where(z > 0, z, 0.2 * z)
        z = jnp.dot(z, w2_ref[...], preferred_element_type=jnp.float32)
        z = z + b2_ref[...]
        z = jnp.where(z > 0, z, 0.2 * z)
        z = jnp.dot(z, w3_ref[...], preferred_element_type=jnp.float32)
        out_ref[...] = z + b3_ref[...]

    return pl.pallas_call(
        body, out_shape=jax.ShapeDtypeStruct((G, 1), jnp.float32))


def kernel(x, edge_index, edge_attr, batch, params):
    N, D = x.shape
    E, DE = edge_attr.shape
    G = 64

    p = params
    src = edge_index[0]
    dst = edge_index[1]
    zeros = jnp.zeros((N, D), jnp.float32)

    wstack = jnp.stack([p["We0"], p["We1"], p["We2"], p["We3"]])
    bstack = jnp.stack([p["be0"], p["be1"], p["be2"], p["be3"]])
    e_all = _build_edge_embed(E, DE, D, 8000)(edge_attr, wstack, bstack)

    sc_agg = _build_sc_agg(N, E, D, 80)
    dense = _build_dense(N, D)

    h = x
    for i in range(4):
        agg_p = sc_agg(h, e_all[i], src, dst, zeros)
        h = dense(
            h, agg_p,
            p[f"W1{i}"], p[f"b1{i}"].reshape(1, D),
            p[f"W2{i}"], p[f"b2{i}"].reshape(1, D),
            p[f"g{i}"].reshape(1, D), p[f"bt{i}"].reshape(1, D),
        )

    score = _build_pool(N, D, G)(
        h, batch.reshape(1, N),
        p["Wm1"], p["bm1"].reshape(1, -1),
        p["Wm2"], p["bm2"].reshape(1, -1),
        p["Wm3"], p["bm3"].reshape(1, -1),
    )
    return score


# speedup vs baseline: 2.7286x; 1.0002x over previous
import functools

import jax
import jax.numpy as jnp
from jax import lax
from jax.experimental import pallas as pl
from jax.experimental.pallas import tpu as pltpu
from jax.experimental.pallas import tpu_sc as plsc

_NC = 2
_NS = 16


def _bf16_rtne(a):
    u = lax.bitcast_convert_type(a, jnp.uint32)
    r = (u + jnp.uint32(0x7FFF) + ((u >> 16) & jnp.uint32(1)))
    return lax.bitcast_convert_type(r & jnp.uint32(0xFFFF0000), jnp.float32)


def _dot_xla_default(a, b):
    return jnp.dot(_bf16_rtne(a), _bf16_rtne(b),
                   preferred_element_type=jnp.float32,
                   precision=lax.Precision.HIGHEST)


def _pack_rows(z):
    zu = lax.bitcast_convert_type(z, jnp.uint32) + jnp.uint32(0x8000)
    lo = zu >> 16
    hi = zu & jnp.uint32(0xFFFF0000)
    parts = []
    for g in range(z.shape[1] // 32):
        parts.append(lo[:, 32 * g:32 * g + 16] | hi[:, 32 * g + 16:32 * g + 32])
    return lax.bitcast_convert_type(jnp.concatenate(parts, axis=1), jnp.int32)


@functools.lru_cache(maxsize=None)
def _build_sc_agg(N, E, D, K):
    NW = _NC * _NS
    EPT = E // NW
    NCH = EPT // K
    NPT = (N // (8 * _NS)) * 8
    REM = N - NPT * _NS
    assert EPT * NW == E and NCH * K == EPT and REM % 8 == 0 and K % 8 == 0
    assert NCH >= 9
    E2 = E // _NC
    MAXOFF = E - K

    mesh = plsc.VectorSubcoreMesh(core_axis_name="c", subcore_axis_name="s")

    @functools.partial(
        pl.kernel,
        mesh=mesh,
        out_type=jax.ShapeDtypeStruct((_NC * N, D), jnp.float32),
    scratch_types=(
            [pltpu.VMEM((K,), jnp.int32)] * 8
            + [pltpu.VMEM((K, D), jnp.float32)] * 4
            + [pltpu.VMEM((K, D // 2), jnp.int32)] * 4
            + [pltpu.VMEM_SHARED((N, D), jnp.float32)]
            + [pltpu.SemaphoreType.DMA] * 20
        ),
    )
    def sc_agg(h_hbm, e_hbm, src_hbm, dst_hbm, zero_hbm, out_hbm,
               is0, is1, is2, is3, id0, id1, id2, id3,
               hb0, hb1, hb2, hb3, eb0, eb1, eb2, eb3, agg_sh,
               ssi0, ssi1, ssi2, ssi3, sdi0, sdi1, sdi2, sdi3,
               se0, se1, se2, se3, sg0, sg1, sg2, sg3, ss0, ss1, ss2, ss3):
        idx_s = (is0, is1, is2, is3)
        idx_d = (id0, id1, id2, id3)
        hbuf = (hb0, hb1, hb2, hb3)
        ebuf = (eb0, eb1, eb2, eb3)
        s_si = (ssi0, ssi1, ssi2, ssi3)
        s_di = (sdi0, sdi1, sdi2, sdi3)
        s_e = (se0, se1, se2, se3)
        s_g = (sg0, sg1, sg2, sg3)
        s_s = (ss0, ss1, ss2, ss3)
        c = lax.axis_index("c")
        s = lax.axis_index("s")

        pltpu.sync_copy(zero_hbm.at[pl.ds(s * NPT, NPT)],
                        agg_sh.at[pl.ds(s * NPT, NPT)])
        if REM:
            @pl.when(s == _NS - 1)
            def _():
                pltpu.sync_copy(zero_hbm.at[pl.ds(_NS * NPT, REM)],
                                agg_sh.at[pl.ds(_NS * NPT, REM)])

        base = c * E2 + s * EPT

        def off_of(t):
            return jnp.minimum(base + t * K, MAXOFF)

        def issue_idx_s(t, j):
            pltpu.async_copy(src_hbm.at[pl.ds(off_of(t), K)], idx_s[j], s_si[j])

        def issue_idx_d(t, j):
            pltpu.async_copy(dst_hbm.at[pl.ds(off_of(t), K)], idx_d[j], s_di[j])

        def issue_e(t, j):
            pltpu.async_copy(e_hbm.at[pl.ds(off_of(t), K)], ebuf[j], s_e[j])

        def wait_idx_s(j):
            pltpu.make_async_copy(src_hbm.at[pl.ds(0, K)], idx_s[j],
                                  s_si[j]).wait()

        def wait_idx_d(j):
            pltpu.make_async_copy(dst_hbm.at[pl.ds(0, K)], idx_d[j],
                                  s_di[j]).wait()

        def wait_e(j):
            pltpu.make_async_copy(e_hbm.at[pl.ds(0, K)], ebuf[j],
                                  s_e[j]).wait()

        def issue_gather(j):
            pltpu.async_copy(h_hbm.at[idx_s[j]], hbuf[j], s_g[j])

        def wait_gather(j):
            pltpu.make_async_copy(h_hbm.at[idx_s[j]], hbuf[j], s_g[j]).wait()

        def issue_scatter(j):
            pltpu.async_copy(hbuf[j], agg_sh.at[idx_d[j]], s_s[j], add=True)

        def wait_scatter(j):
            pltpu.make_async_copy(hbuf[j], agg_sh.at[idx_d[j]], s_s[j]).wait()

        def compute(j):
            def row(r, cc):
                for g in range(D // 32):
                    xe = ebuf[j][r, pl.ds(16 * g, 16)]
                    e_lo = lax.bitcast_convert_type(xe << 16, jnp.float32)
                    e_hi = lax.bitcast_convert_type(xe & -65536, jnp.float32)
                    slo = pl.ds(32 * g, 16)
                    shi = pl.ds(32 * g + 16, 16)
                    hbuf[j][r, slo] = jnp.maximum(hbuf[j][r, slo] + e_lo, 0.0)
                    hbuf[j][r, shi] = jnp.maximum(hbuf[j][r, shi] + e_hi, 0.0)
                return cc
            lax.fori_loop(0, K, row, 0, unroll=4)

        def step(t, j, first=False):
            jm1 = (j + 3) % 4
            wait_gather(j)
            wait_e(j)
            compute(j)
            wait_idx_d(j)
            issue_scatter(j)
            issue_idx_s(t + 3, jm1)
            issue_e(t + 3, jm1)
            if not first:
                wait_scatter(jm1)
            issue_idx_d(t + 3, jm1)
            wait_idx_s(jm1)
            issue_gather(jm1)

        for t0 in range(3):
            issue_idx_s(t0, t0)
            issue_idx_d(t0, t0)
            issue_e(t0, t0)
        for t0 in range(3):
            wait_idx_s(t0)
            issue_gather(t0)
        plsc.subcore_barrier()

        step(0, 0, first=True)

        def quad(i, cc):
            t = 1 + 4 * i
            for k in range(4):
                step(t + k, (1 + k) % 4)
            return cc

        lax.fori_loop(0, (NCH - 1) // 4, quad, 0)
        for t1 in range(NCH - ((NCH - 1) % 4), NCH):
            step(t1, t1 % 4)

        wait_scatter((NCH - 1) % 4)
        for k in range(1, 4):
            jd = (NCH - 1 + k) % 4
            wait_gather(jd)
            wait_e(jd)
            wait_idx_d(jd)
        plsc.subcore_barrier()

        pltpu.sync_copy(agg_sh.at[pl.ds(s * NPT, NPT)],
                        out_hbm.at[pl.ds(c * N + s * NPT, NPT)])
        if REM:
            @pl.when(s == _NS - 1)
            def _():
                pltpu.sync_copy(agg_sh.at[pl.ds(_NS * NPT, REM)],
                                out_hbm.at[pl.ds(c * N + _NS * NPT, REM)])

    return sc_agg


@functools.lru_cache(maxsize=None)
def _build_edge_embed(E, DE, D, EB):
    grid = (E // EB,)

    def body(ea_ref, w_ref, b_ref, o0, o1, o2, o3):
        ea = ea_ref[...]
        outs = (o0, o1, o2, o3)
        for i in range(4):
            e = (jnp.dot(ea.astype(jnp.bfloat16),
                         w_ref[i].astype(jnp.bfloat16),
                         preferred_element_type=jnp.float32)
                 + b_ref[i, :][None, :])
            outs[i][...] = _pack_rows(e)

    return pl.pallas_call(
        body,
        grid=grid,
        in_specs=[
            pl.BlockSpec((EB, DE), lambda i: (i, 0)),
            pl.BlockSpec((4, DE, D), lambda i: (0, 0, 0)),
            pl.BlockSpec((4, D), lambda i: (0, 0)),
        ],
        out_specs=[pl.BlockSpec((EB, D // 2), lambda i: (i, 0))] * 4,
        out_shape=[jax.ShapeDtypeStruct((E, D // 2), jnp.int32)] * 4,
    )


@functools.lru_cache(maxsize=None)
def _build_dense(N, D):
    def body(h_ref, ap_ref, w1_ref, b1_ref, w2_ref, b2_ref, g_ref, bt_ref,
             out_ref):
        ap = ap_ref[...]
        z = h_ref[...] + ap[:N] + ap[N:]
        z = _dot_xla_default(z, w1_ref[...])
        z = jnp.maximum(z + b1_ref[...], 0.0)
        z = _dot_xla_default(z, w2_ref[...])
        z = z + b2_ref[...]
        mu = jnp.mean(z, axis=0, keepdims=True)
        zc = z - mu
        var = jnp.mean(zc * zc, axis=0, keepdims=True)
        zn = zc / jnp.sqrt(var + 1e-5) * g_ref[...] + bt_ref[...]
        out_ref[...] = jnp.maximum(zn, 0.0)

    return pl.pallas_call(
        body, out_shape=jax.ShapeDtypeStruct((N, D), jnp.float32))


@functools.lru_cache(maxsize=None)
def _build_pool(N, D, G):
    def body(h_ref, b_ref, w1_ref, b1_ref, w2_ref, b2_ref, w3_ref, b3_ref,
             out_ref):
        gids = lax.broadcasted_iota(jnp.int32, (G, N), 0)
        onehot = (gids == b_ref[...]).astype(jnp.float32)
        sums = jnp.dot(onehot, h_ref[...], preferred_element_type=jnp.float32, precision=lax.Precision.HIGHEST)
        cnt = jnp.sum(onehot, axis=1, keepdims=True)
        gm = sums / jnp.maximum(cnt, 1.0)
        z = _dot_xla_default(gm, w1_ref[...])
        z = z + b1_ref[...]
        z = jnp.where(z > 0, z, 0.2 * z)
        z = _dot_xla_default(z, w2_ref[...])
        z = z + b2_ref[...]
        z = jnp.where(z > 0, z, 0.2 * z)
        z = _dot_xla_default(z, w3_ref[...])
        out_ref[...] = z + b3_ref[...]

    return pl.pallas_call(
        body, out_shape=jax.ShapeDtypeStruct((G, 1), jnp.float32))


def kernel(x, edge_index, edge_attr, batch, params):
    N, D = x.shape
    E, DE = edge_attr.shape
    G = 64

    p = params
    src = edge_index[0]
    dst = edge_index[1]
    zeros = jnp.zeros((N, D), jnp.float32)

    wstack = jnp.stack([p["We0"], p["We1"], p["We2"], p["We3"]])
    bstack = jnp.stack([p["be0"], p["be1"], p["be2"], p["be3"]])
    e_all = _build_edge_embed(E, DE, D, 8000)(edge_attr, wstack, bstack)

    sc_agg = _build_sc_agg(N, E, D, 40)
    dense = _build_dense(N, D)

    h = x
    for i in range(4):
        agg_p = sc_agg(h, e_all[i], src, dst, zeros)
        h = dense(
            h, agg_p,
            p[f"W1{i}"], p[f"b1{i}"].reshape(1, D),
            p[f"W2{i}"], p[f"b2{i}"].reshape(1, D),
            p[f"g{i}"].reshape(1, D), p[f"bt{i}"].reshape(1, D),
        )

    score = _build_pool(N, D, G)(
        h, batch.reshape(1, N),
        p["Wm1"], p["bm1"].reshape(1, -1),
        p["Wm2"], p["bm2"].reshape(1, -1),
        p["Wm3"], p["bm3"].reshape(1, -1),
    )
    return score
